# baseline (speedup 1.0000x reference)
import functools

import jax
import jax.numpy as jnp
from jax import lax
from jax.experimental import pallas as pl
from jax.experimental.pallas import tpu as pltpu
from jax.experimental.pallas import tpu_sc as plsc

N = 10000
NP = 10240
E = 320000
EP = 331776
HID = 128
ROWW = 144
NG = 64
IN_DIM = 4527
HOG = 4464
CH = 64


def _k1_body(x_ref, cov_ref, wcat_ref, wcov_ref, att_ref, bd_ref,
             xt_ref, tabs_ref, m_ref, macc_ref):
    i = pl.program_id(0)
    xb = x_ref[...]
    p = jnp.dot(xb, wcat_ref[...], preferred_element_type=jnp.float32)
    vv = jnp.dot(cov_ref[...], wcov_ref[...],
                 preferred_element_type=jnp.float32)
    q = p[:, 256:384]
    lg = q[:, 0:1] + bd_ref[0, 0]
    aw0 = 1.0 / (1.0 + jnp.exp(lg))
    xtb = aw0 * p[:, 0:256] + (1.0 - 2.0 * aw0) * vv
    row = i * 256 + lax.broadcasted_iota(jnp.int32, (256, 1), 0)
    xtb = jnp.where(row < N, xtb, 0.0)
    t0 = lax.dot_general(att_ref[0], xtb[:, 0:128], (((1,), (1,)), ((), ())),
                         preferred_element_type=jnp.float32)
    t1 = lax.dot_general(att_ref[1], xtb[:, 128:256], (((1,), (1,)), ((), ())),
                         preferred_element_type=jnp.float32)
    tabs_ref[...] = jnp.concatenate(
        [t0[1:2], t1[1:2], jnp.zeros((6, 256), jnp.float32)], axis=0)
    zero15 = jnp.zeros((256, 15), jnp.float32)
    xt_ref[...] = jnp.concatenate(
        [jnp.concatenate([xtb[:, 0:128],
                          jnp.sum(xtb[:, 0:128] * att_ref[0, 0:1, :], axis=1,
                                  keepdims=True), zero15],
                         axis=1).reshape(1, 256, ROWW),
         jnp.concatenate([xtb[:, 128:256],
                          jnp.sum(xtb[:, 128:256] * att_ref[1, 0:1, :], axis=1,
                                  keepdims=True), zero15],
                         axis=1).reshape(1, 256, ROWW)], axis=0)
    tcat = jnp.concatenate([t0[0:2], t1[0:2],
                            jnp.zeros((4, 256), jnp.float32)], axis=0)

    @pl.when(i == 0)
    def _():
        macc_ref[...] = jnp.full((8, 256), -1e30, jnp.float32)
    macc_ref[...] = jnp.maximum(macc_ref[...], tcat)

    @pl.when(i == 39)
    def _():
        mm = jnp.max(macc_ref[...], axis=1, keepdims=True)
        m0 = mm[0:1] + mm[1:2]
        m1 = mm[2:3] + mm[3:4]
        m0 = jnp.where(m0 >= 0.0, m0, 0.2 * m0)
        m1 = jnp.where(m1 >= 0.0, m1, 0.2 * m1)
        m_ref[...] = jnp.concatenate(
            [jnp.broadcast_to(m0, (1, 128)), jnp.broadcast_to(m1, (1, 128)),
             jnp.zeros((6, 128), jnp.float32)], axis=0)


def _run_k1(x, covp, wcat, wcov, att, bd, interpret=False):
    return pl.pallas_call(
        _k1_body,
        grid=(40,),
        in_specs=[
            pl.BlockSpec((256, IN_DIM), lambda i: (i, 0)),
            pl.BlockSpec((256, 128), lambda i: (i, 0)),
            pl.BlockSpec((IN_DIM, 384), lambda i: (0, 0)),
            pl.BlockSpec((128, 256), lambda i: (0, 0)),
            pl.BlockSpec((2, 8, 128), lambda i: (0, 0, 0)),
            pl.BlockSpec((1, 1), lambda i: (0, 0)),
        ],
        out_specs=[
            pl.BlockSpec((2, 256, ROWW), lambda i: (0, i, 0)),
            pl.BlockSpec((8, 256), lambda i: (0, i)),
            pl.BlockSpec((8, 128), lambda i: (0, 0)),
        ],
        out_shape=[
            jax.ShapeDtypeStruct((2, NP, ROWW), jnp.float32),
            jax.ShapeDtypeStruct((8, NP), jnp.float32),
            jax.ShapeDtypeStruct((8, 128), jnp.float32),
        ],
        scratch_shapes=[pltpu.VMEM((8, 256), jnp.float32)],
        interpret=interpret,
    )(x, covp, wcat, wcov, att, bd)


@functools.lru_cache(maxsize=None)
def _make_edge_kernel(head_split):
    n_tiles = 16 if head_split else 32
    nchunks = EP // (n_tiles * CH)
    mesh = plsc.VectorSubcoreMesh(core_axis_name="c", subcore_axis_name="s",
                                  num_cores=2, num_subcores=16)

    def body(xt_hbm, src_hbm, dst_hbm, tabs_hbm, m_hbm, out_hbm,
             adst_t, mvb, srcb, dstb, idxb, sdst, exb, gbuf, obuf, acc_sh,
             gsem, ssem):
        cid = lax.axis_index("c")
        sid = lax.axis_index("s")
        if head_split:
            adst_row = cid
            m_row = cid
            tile_base = sid * (nchunks * CH)
            goff = cid * NP
        else:
            adst_row = 0
            m_row = 0
            wid = sid * 2 + cid
            tile_base = wid * (nchunks * CH)
            goff = 0

        pltpu.sync_copy(tabs_hbm.at[adst_row], adst_t)
        pltpu.sync_copy(m_hbm.at[m_row, pl.ds(0, 16)], mvb)
        mv = mvb[...]

        def zbody(r, _):
            for q in range(ROWW // 16):
                obuf[r, pl.ds(q * 16, 16)] = jnp.zeros((16,), jnp.float32)
            return 0
        lax.fori_loop(0, 32, zbody, 0)
        rows_per_tile = NP // 16
        for r in range(rows_per_tile // 32):
            pltpu.sync_copy(obuf.at[pl.ds(0, 32)],
                            acc_sh.at[pl.ds(sid * rows_per_tile + r * 32, 32)])
        plsc.subcore_barrier()

        lane = lax.iota(jnp.int32, 16)
        col128 = jnp.full((16,), 128, jnp.int32)

        def stage_and_gather(c, b):
            base = tile_base + c * CH
            pltpu.sync_copy(src_hbm.at[pl.ds(base, CH)], srcb.at[b])
            pltpu.sync_copy(dst_hbm.at[pl.ds(base, CH)], dstb.at[b])
            for j in range(CH // 16):
                idxb[b, pl.ds(j * 16, 16)] = (srcb[b, pl.ds(j * 16, 16)]
                                              + goff)
            pltpu.async_copy(xt_hbm.at[idxb.at[b]], gbuf.at[b], gsem)

        def wait_gather(b):
            pltpu.make_async_copy(xt_hbm.at[idxb.at[b]], gbuf.at[b],
                                  gsem).wait()

        def wait_scatter():
            pltpu.make_async_copy(obuf, acc_sh.at[sdst], ssem).wait()

        stage_and_gather(0, 0)

        def chunk_body(c, _):
            b = lax.rem(c, 2)
            nb = 1 - b
            wait_gather(b)

            @pl.when(c + 1 < nchunks)
            def _():
                stage_and_gather(c + 1, nb)

            for j in range(CH // 16):
                el16 = lane + j * 16
                d16 = dstb[b, pl.ds(j * 16, 16)]
                a = (plsc.load_gather(gbuf.at[b], [el16, col128])
                     + plsc.load_gather(adst_t, [d16]))
                a = jnp.where(a >= 0.0, a, 0.2 * a)
                exb[pl.ds(j * 16, 16)] = jnp.exp(a - mv)

            @pl.when(c >= 1)
            def _():
                wait_scatter()
            for j in range(CH // 16):
                sdst[pl.ds(j * 16, 16)] = dstb[b, pl.ds(j * 16, 16)]

            def erow(j, _):
                exv = exb[pl.ds(j * 16, 16)]
                for t in range(16):
                    e = j * 16 + t
                    sp = jnp.full((16,), exv[t])
                    for q in range(8):
                        obuf[e, pl.ds(q * 16, 16)] = (
                            gbuf[b, e, pl.ds(q * 16, 16)] * sp)
                    obuf[e, pl.ds(128, 16)] = jnp.where(lane == 0, sp, 0.0)
                return 0
            lax.fori_loop(0, CH // 16, erow, 0)
            pltpu.async_copy(obuf, acc_sh.at[sdst], ssem, add=True)
            return 0

        lax.fori_loop(0, nchunks, chunk_body, 0)
        wait_scatter()
        plsc.subcore_barrier()
        pltpu.sync_copy(acc_sh.at[pl.ds(sid * rows_per_tile, rows_per_tile)],
                        out_hbm.at[cid, pl.ds(sid * rows_per_tile,
                                              rows_per_tile)])

    return pl.kernel(
        body,
        out_type=jax.ShapeDtypeStruct((2, NP, ROWW), jnp.float32),
        mesh=mesh,
        compiler_params=pltpu.CompilerParams(needs_layout_passes=False,
                                             use_tc_tiling_on_sc=False,
                                             internal_scratch_in_bytes=32768),
        scratch_types=[
            pltpu.VMEM((NP,), jnp.float32),
            pltpu.VMEM((16,), jnp.float32),
            pltpu.VMEM((2, CH), jnp.int32),
            pltpu.VMEM((2, CH), jnp.int32),
            pltpu.VMEM((2, CH), jnp.int32),
            pltpu.VMEM((CH,), jnp.int32),
            pltpu.VMEM((CH,), jnp.float32),
            pltpu.VMEM((2, CH, ROWW), jnp.float32),
            pltpu.VMEM((CH, ROWW), jnp.float32),
            pltpu.VMEM_SHARED((NP, ROWW), jnp.float32),
            pltpu.SemaphoreType.DMA,
            pltpu.SemaphoreType.DMA,
        ],
    )


def _k3_body(s_ref, w2_ref, b1_ref, att2_ref, xt2_ref, tabs2_ref, m2_ref,
             macc_ref):
    i = pl.program_id(0)
    s = s_ref[...]
    row = i * 512 + lax.broadcasted_iota(jnp.int32, (512, 1), 0)
    ok = row < N

    def head(h):
        d = s[h, :, 128:144][:, 0:1] + 1e-16
        hh = jnp.maximum(s[h, :, 0:128] / d + b1_ref[h:h + 1, :], 0.0)
        return jnp.where(ok, hh, 0.0)
    h0 = head(0)
    h1 = head(1)
    xt2b = (jnp.dot(h0, w2_ref[0:128, :], preferred_element_type=jnp.float32)
            + jnp.dot(h1, w2_ref[128:256, :],
                      preferred_element_type=jnp.float32))
    t = lax.dot_general(att2_ref[...], xt2b, (((1,), (1,)), ((), ())),
                        preferred_element_type=jnp.float32)
    tabs2_ref[...] = jnp.concatenate(
        [t[1:2], jnp.zeros((7, 512), jnp.float32)], axis=0)
    asrc2 = jnp.sum(xt2b * att2_ref[0:1, :], axis=1, keepdims=True)
    xt2_ref[...] = jnp.concatenate(
        [xt2b, asrc2, jnp.zeros((512, 15), jnp.float32)], axis=1)

    @pl.when(i == 0)
    def _():
        macc_ref[...] = jnp.full((8, 512), -1e30, jnp.float32)
    macc_ref[...] = jnp.maximum(macc_ref[...], t)

    @pl.when(i == 19)
    def _():
        mm = jnp.max(macc_ref[...], axis=1, keepdims=True)
        m0 = mm[0:1] + mm[1:2]
        m0 = jnp.where(m0 >= 0.0, m0, 0.2 * m0)
        m2_ref[...] = jnp.concatenate(
            [jnp.broadcast_to(m0, (1, 128)),
             jnp.zeros((7, 128), jnp.float32)], axis=0)


def _run_k3(s1, w2, b1r, att2, interpret=False):
    return pl.pallas_call(
        _k3_body,
        grid=(20,),
        in_specs=[
            pl.BlockSpec((2, 512, ROWW), lambda i: (0, i, 0)),
            pl.BlockSpec((256, 128), lambda i: (0, 0)),
            pl.BlockSpec((2, 128), lambda i: (0, 0)),
            pl.BlockSpec((8, 128), lambda i: (0, 0)),
        ],
        out_specs=[
            pl.BlockSpec((512, ROWW), lambda i: (i, 0)),
            pl.BlockSpec((8, 512), lambda i: (0, i)),
            pl.BlockSpec((8, 128), lambda i: (0, 0)),
        ],
        out_shape=[
            jax.ShapeDtypeStruct((NP, ROWW), jnp.float32),
            jax.ShapeDtypeStruct((8, NP), jnp.float32),
            jax.ShapeDtypeStruct((8, 128), jnp.float32),
        ],
        scratch_shapes=[pltpu.VMEM((8, 512), jnp.float32)],
        interpret=interpret,
    )(s1, w2, b1r, att2)


def _k5_body(s_ref, bh_ref, b2_ref, cw1_ref, cb1_ref, cw2_ref, cb2_ref,
             out_ref, acc_ref):
    i = pl.program_id(0)
    s = s_ref[0] + s_ref[1]
    row = i * 512 + lax.broadcasted_iota(jnp.int32, (512, 1), 0)
    ok = row < N
    d = s[:, 128:144][:, 0:1] + 1e-16
    h2 = jnp.maximum(s[:, 0:128] / d + b2_ref[...], 0.0)
    h2 = jnp.where(ok, h2, 0.0)
    lane = lax.broadcasted_iota(jnp.int32, (512, 16), 1)
    ones = jnp.where(ok & (lane == 0), 1.0, 0.0)
    aug = jnp.concatenate([h2, ones], axis=1)
    contrib = lax.dot_general(bh_ref[...], aug, (((0,), (0,)), ((), ())),
                              preferred_element_type=jnp.float32)

    @pl.when(i == 0)
    def _():
        acc_ref[...] = contrib

    @pl.when(i > 0)
    def _():
        acc_ref[...] = acc_ref[...] + contrib

    @pl.when(i == 19)
    def _():
        a = acc_ref[...]
        pooled = a[:, 0:128] / jnp.maximum(a[:, 128:144][:, 0:1], 1.0)
        z = jnp.maximum(jnp.dot(pooled, cw1_ref[...],
                                preferred_element_type=jnp.float32)
                        + cb1_ref[...], 0.0)
        out_ref[...] = (jnp.dot(z, cw2_ref[...],
                                preferred_element_type=jnp.float32)
                        + cb2_ref[...])


def _run_k5(s2, bh, b2r, cw1, cb1r, cw2, cb2r, interpret=False):
    return pl.pallas_call(
        _k5_body,
        grid=(20,),
        in_specs=[
            pl.BlockSpec((2, 512, ROWW), lambda i: (0, i, 0)),
            pl.BlockSpec((512, NG), lambda i: (i, 0)),
            pl.BlockSpec((1, 128), lambda i: (0, 0)),
            pl.BlockSpec((128, NG), lambda i: (0, 0)),
            pl.BlockSpec((1, NG), lambda i: (0, 0)),
            pl.BlockSpec((NG, 4), lambda i: (0, 0)),
            pl.BlockSpec((1, 4), lambda i: (0, 0)),
        ],
        out_specs=pl.BlockSpec((NG, 4), lambda i: (0, 0)),
        out_shape=jax.ShapeDtypeStruct((NG, 4), jnp.float32),
        scratch_shapes=[pltpu.VMEM((NG, ROWW), jnp.float32)],
        interpret=interpret,
    )(s2, bh, b2r, cw1, cb1r, cw2, cb2r)


def kernel(x, edge_index, batch, attn_W, attn_b, W1, att_src1, att_dst1, b1,
           W2, att_src2, att_dst2, b2, cW1, cb1, cW2, cb2):
    f32 = jnp.float32
    x = x.astype(f32)
    ei = edge_index.astype(jnp.int32)
    batch32 = batch.astype(jnp.int32)

    delta = (attn_W[:, 1] - attn_W[:, 0]).reshape(IN_DIM, 1)
    wcat = jnp.concatenate(
        [W1, delta, jnp.zeros((IN_DIM, 127), f32)], axis=1)
    wcov = jnp.concatenate(
        [W1[HOG:], jnp.zeros((128 - (IN_DIM - HOG), 256), f32)], axis=0)
    covp = jnp.pad(x[:, HOG:], ((0, 0), (0, 128 - (IN_DIM - HOG))))
    att = jnp.stack([
        jnp.concatenate([att_src1[0:1, :], att_dst1[0:1, :],
                         jnp.zeros((6, 128), f32)], axis=0),
        jnp.concatenate([att_src1[1:2, :], att_dst1[1:2, :],
                         jnp.zeros((6, 128), f32)], axis=0)])
    bd = (attn_b[1] - attn_b[0]).reshape(1, 1)

    loop = jnp.arange(N, dtype=jnp.int32)
    src = jnp.concatenate([ei[0], loop, jnp.zeros((EP - E - N,), jnp.int32)])
    dst = jnp.concatenate([ei[1], loop,
                           jnp.full((EP - E - N,), N, jnp.int32)])

    att2 = jnp.concatenate([att_src2[0:1, :], att_dst2[0:1, :],
                            jnp.zeros((6, 128), f32)], axis=0)
    b1r = b1.reshape(2, 128)
    b2r = b2.reshape(1, 128)
    cb1r = cb1.reshape(1, NG)
    cb2r = cb2.reshape(1, 4)
    bh = (jnp.pad(batch32, (0, NP - N), constant_values=NG)[:, None]
          == jnp.arange(NG, dtype=jnp.int32)[None, :]).astype(f32)

    xt_heads, tabs1, m1 = _run_k1(x, covp, wcat, wcov, att, bd)
    xt_cat = xt_heads.reshape(2 * NP, ROWW)

    s1 = _make_edge_kernel(True)(xt_cat, src, dst, tabs1, m1)

    xt2, tabs2, m2 = _run_k3(s1, W2, b1r, att2)

    s2 = _make_edge_kernel(False)(xt2, src, dst, tabs2, m2)

    return _run_k5(s2, bh, b2r, cW1, cb1r, cW2, cb2r)


# speedup vs baseline: 1.0406x; 1.0406x over previous
import functools

import jax
import jax.numpy as jnp
from jax import lax
from jax.experimental import pallas as pl
from jax.experimental.pallas import tpu as pltpu
from jax.experimental.pallas import tpu_sc as plsc

N = 10000
NP = 10240
E = 320000
EP = 331776
HID = 128
ROWW = 144
NG = 64
IN_DIM = 4527
HOG = 4464
CH = 64


def _k1_body(x_ref, cov_ref, wcat_ref, wcov_ref, att_ref, bd_ref,
             xt_ref, tabs_ref, m_ref, macc_ref):
    i = pl.program_id(0)
    xb = x_ref[...]
    p = jnp.dot(xb, wcat_ref[...], preferred_element_type=jnp.float32)
    vv = jnp.dot(cov_ref[...], wcov_ref[...],
                 preferred_element_type=jnp.float32)
    q = p[:, 256:384]
    lg = q[:, 0:1] + bd_ref[0, 0]
    aw0 = 1.0 / (1.0 + jnp.exp(lg))
    xtb = aw0 * p[:, 0:256] + (1.0 - 2.0 * aw0) * vv
    row = i * 256 + lax.broadcasted_iota(jnp.int32, (256, 1), 0)
    xtb = jnp.where(row < N, xtb, 0.0)
    t0 = lax.dot_general(att_ref[0], xtb[:, 0:128], (((1,), (1,)), ((), ())),
                         preferred_element_type=jnp.float32)
    t1 = lax.dot_general(att_ref[1], xtb[:, 128:256], (((1,), (1,)), ((), ())),
                         preferred_element_type=jnp.float32)
    tabs_ref[...] = jnp.concatenate(
        [t0[1:2], t1[1:2], jnp.zeros((6, 256), jnp.float32)], axis=0)
    zero15 = jnp.zeros((256, 15), jnp.float32)
    xt_ref[...] = jnp.concatenate(
        [jnp.concatenate([xtb[:, 0:128],
                          jnp.sum(xtb[:, 0:128] * att_ref[0, 0:1, :], axis=1,
                                  keepdims=True), zero15],
                         axis=1).reshape(1, 256, ROWW),
         jnp.concatenate([xtb[:, 128:256],
                          jnp.sum(xtb[:, 128:256] * att_ref[1, 0:1, :], axis=1,
                                  keepdims=True), zero15],
                         axis=1).reshape(1, 256, ROWW)], axis=0)
    tcat = jnp.concatenate([t0[0:2], t1[0:2],
                            jnp.zeros((4, 256), jnp.float32)], axis=0)

    @pl.when(i == 0)
    def _():
        macc_ref[...] = jnp.full((8, 256), -1e30, jnp.float32)
    macc_ref[...] = jnp.maximum(macc_ref[...], tcat)

    @pl.when(i == 39)
    def _():
        mm = jnp.max(macc_ref[...], axis=1, keepdims=True)
        m0 = mm[0:1] + mm[1:2]
        m1 = mm[2:3] + mm[3:4]
        m0 = jnp.where(m0 >= 0.0, m0, 0.2 * m0)
        m1 = jnp.where(m1 >= 0.0, m1, 0.2 * m1)
        m_ref[...] = jnp.concatenate(
            [jnp.broadcast_to(m0, (1, 128)), jnp.broadcast_to(m1, (1, 128)),
             jnp.zeros((6, 128), jnp.float32)], axis=0)


def _run_k1(x, covp, wcat, wcov, att, bd, interpret=False):
    return pl.pallas_call(
        _k1_body,
        grid=(40,),
        in_specs=[
            pl.BlockSpec((256, IN_DIM), lambda i: (i, 0)),
            pl.BlockSpec((256, 128), lambda i: (i, 0)),
            pl.BlockSpec((IN_DIM, 384), lambda i: (0, 0)),
            pl.BlockSpec((128, 256), lambda i: (0, 0)),
            pl.BlockSpec((2, 8, 128), lambda i: (0, 0, 0)),
            pl.BlockSpec((1, 1), lambda i: (0, 0)),
        ],
        out_specs=[
            pl.BlockSpec((2, 256, ROWW), lambda i: (0, i, 0)),
            pl.BlockSpec((8, 256), lambda i: (0, i)),
            pl.BlockSpec((8, 128), lambda i: (0, 0)),
        ],
        out_shape=[
            jax.ShapeDtypeStruct((2, NP, ROWW), jnp.float32),
            jax.ShapeDtypeStruct((8, NP), jnp.float32),
            jax.ShapeDtypeStruct((8, 128), jnp.float32),
        ],
        scratch_shapes=[pltpu.VMEM((8, 256), jnp.float32)],
        interpret=interpret,
    )(x, covp, wcat, wcov, att, bd)


@functools.lru_cache(maxsize=None)
def _make_edge_kernel(head_split):
    n_tiles = 16 if head_split else 32
    nchunks = EP // (n_tiles * CH)
    mesh = plsc.VectorSubcoreMesh(core_axis_name="c", subcore_axis_name="s",
                                  num_cores=2, num_subcores=16)

    def body(xt_hbm, src_hbm, dst_hbm, tabs_hbm, m_hbm, out_hbm,
             adst_t, mvb, srcb, dstb, idxb, exb, gbuf, obuf, acc_sh, gsem):
        cid = lax.axis_index("c")
        sid = lax.axis_index("s")
        if head_split:
            adst_row = cid
            m_row = cid
            tile_base = sid * (nchunks * CH)
            goff = cid * NP
        else:
            adst_row = 0
            m_row = 0
            wid = sid * 2 + cid
            tile_base = wid * (nchunks * CH)
            goff = 0

        pltpu.sync_copy(tabs_hbm.at[adst_row], adst_t)
        pltpu.sync_copy(m_hbm.at[m_row, pl.ds(0, 16)], mvb)
        mv = mvb[...]

        def zbody(r, _):
            for q in range(ROWW // 16):
                obuf[r, pl.ds(q * 16, 16)] = jnp.zeros((16,), jnp.float32)
            return 0
        lax.fori_loop(0, 32, zbody, 0)
        rows_per_tile = NP // 16
        for r in range(rows_per_tile // 32):
            pltpu.sync_copy(obuf.at[pl.ds(0, 32)],
                            acc_sh.at[pl.ds(sid * rows_per_tile + r * 32, 32)])
        plsc.subcore_barrier()

        lane = lax.iota(jnp.int32, 16)
        col128 = jnp.full((16,), 128, jnp.int32)

        def chunk_body(c, _):
            base = tile_base + c * CH
            pltpu.sync_copy(src_hbm.at[pl.ds(base, CH)], srcb)
            pltpu.sync_copy(dst_hbm.at[pl.ds(base, CH)], dstb)

            def oidx(j, _):
                idxb[pl.ds(j * 16, 16)] = srcb[pl.ds(j * 16, 16)] + goff
                return 0
            lax.fori_loop(0, CH // 16, oidx, 0)
            pltpu.async_copy(xt_hbm.at[idxb], gbuf, gsem).wait()

            def exg(j, _):
                el16 = lane + j * 16
                d16 = dstb[pl.ds(j * 16, 16)]
                a = (plsc.load_gather(gbuf, [el16, col128])
                     + plsc.load_gather(adst_t, [d16]))
                a = jnp.where(a >= 0.0, a, 0.2 * a)
                exb[pl.ds(j * 16, 16)] = jnp.exp(a - mv)
                return 0
            lax.fori_loop(0, CH // 16, exg, 0)

            def erow(j, _):
                exv = exb[pl.ds(j * 16, 16)]
                for t in range(16):
                    e = j * 16 + t
                    sp = jnp.full((16,), exv[t])
                    for q in range(8):
                        obuf[e, pl.ds(q * 16, 16)] = (
                            gbuf[e, pl.ds(q * 16, 16)] * sp)
                    obuf[e, pl.ds(128, 16)] = jnp.where(lane == 0, sp, 0.0)
                return 0
            lax.fori_loop(0, CH // 16, erow, 0)
            pltpu.sync_copy(obuf, acc_sh.at[dstb], add=True)
            return 0

        lax.fori_loop(0, nchunks, chunk_body, 0)
        plsc.subcore_barrier()

        def drain(r, _):
            r0 = sid * rows_per_tile + r * 64
            pltpu.sync_copy(acc_sh.at[pl.ds(r0, 64)],
                            out_hbm.at[cid, pl.ds(r0, 64)])
            return 0
        lax.fori_loop(0, rows_per_tile // 64, drain, 0)

    return pl.kernel(
        body,
        out_type=jax.ShapeDtypeStruct((2, NP, ROWW), jnp.float32),
        mesh=mesh,
        compiler_params=pltpu.CompilerParams(needs_layout_passes=False,
                                             use_tc_tiling_on_sc=False,
                                             internal_scratch_in_bytes=32768),
        scratch_types=[
            pltpu.VMEM((NP,), jnp.float32),
            pltpu.VMEM((16,), jnp.float32),
            pltpu.VMEM((CH,), jnp.int32),
            pltpu.VMEM((CH,), jnp.int32),
            pltpu.VMEM((CH,), jnp.int32),
            pltpu.VMEM((CH,), jnp.float32),
            pltpu.VMEM((CH, ROWW), jnp.float32),
            pltpu.VMEM((CH, ROWW), jnp.float32),
            pltpu.VMEM_SHARED((NP, ROWW), jnp.float32),
            pltpu.SemaphoreType.DMA,
        ],
    )


def _k3_body(s_ref, w2_ref, b1_ref, att2_ref, xt2_ref, tabs2_ref, m2_ref,
             macc_ref):
    i = pl.program_id(0)
    s = s_ref[...]
    row = i * 512 + lax.broadcasted_iota(jnp.int32, (512, 1), 0)
    ok = row < N

    def head(h):
        d = s[h, :, 128:144][:, 0:1] + 1e-16
        hh = jnp.maximum(s[h, :, 0:128] / d + b1_ref[h:h + 1, :], 0.0)
        return jnp.where(ok, hh, 0.0)
    h0 = head(0)
    h1 = head(1)
    xt2b = (jnp.dot(h0, w2_ref[0:128, :], preferred_element_type=jnp.float32)
            + jnp.dot(h1, w2_ref[128:256, :],
                      preferred_element_type=jnp.float32))
    t = lax.dot_general(att2_ref[...], xt2b, (((1,), (1,)), ((), ())),
                        preferred_element_type=jnp.float32)
    tabs2_ref[...] = jnp.concatenate(
        [t[1:2], jnp.zeros((7, 512), jnp.float32)], axis=0)
    asrc2 = jnp.sum(xt2b * att2_ref[0:1, :], axis=1, keepdims=True)
    xt2_ref[...] = jnp.concatenate(
        [xt2b, asrc2, jnp.zeros((512, 15), jnp.float32)], axis=1)

    @pl.when(i == 0)
    def _():
        macc_ref[...] = jnp.full((8, 512), -1e30, jnp.float32)
    macc_ref[...] = jnp.maximum(macc_ref[...], t)

    @pl.when(i == 19)
    def _():
        mm = jnp.max(macc_ref[...], axis=1, keepdims=True)
        m0 = mm[0:1] + mm[1:2]
        m0 = jnp.where(m0 >= 0.0, m0, 0.2 * m0)
        m2_ref[...] = jnp.concatenate(
            [jnp.broadcast_to(m0, (1, 128)),
             jnp.zeros((7, 128), jnp.float32)], axis=0)


def _run_k3(s1, w2, b1r, att2, interpret=False):
    return pl.pallas_call(
        _k3_body,
        grid=(20,),
        in_specs=[
            pl.BlockSpec((2, 512, ROWW), lambda i: (0, i, 0)),
            pl.BlockSpec((256, 128), lambda i: (0, 0)),
            pl.BlockSpec((2, 128), lambda i: (0, 0)),
            pl.BlockSpec((8, 128), lambda i: (0, 0)),
        ],
        out_specs=[
            pl.BlockSpec((512, ROWW), lambda i: (i, 0)),
            pl.BlockSpec((8, 512), lambda i: (0, i)),
            pl.BlockSpec((8, 128), lambda i: (0, 0)),
        ],
        out_shape=[
            jax.ShapeDtypeStruct((NP, ROWW), jnp.float32),
            jax.ShapeDtypeStruct((8, NP), jnp.float32),
            jax.ShapeDtypeStruct((8, 128), jnp.float32),
        ],
        scratch_shapes=[pltpu.VMEM((8, 512), jnp.float32)],
        interpret=interpret,
    )(s1, w2, b1r, att2)


def _k5_body(s_ref, bh_ref, b2_ref, cw1_ref, cb1_ref, cw2_ref, cb2_ref,
             out_ref, acc_ref):
    i = pl.program_id(0)
    s = s_ref[0] + s_ref[1]
    row = i * 512 + lax.broadcasted_iota(jnp.int32, (512, 1), 0)
    ok = row < N
    d = s[:, 128:144][:, 0:1] + 1e-16
    h2 = jnp.maximum(s[:, 0:128] / d + b2_ref[...], 0.0)
    h2 = jnp.where(ok, h2, 0.0)
    lane = lax.broadcasted_iota(jnp.int32, (512, 16), 1)
    ones = jnp.where(ok & (lane == 0), 1.0, 0.0)
    aug = jnp.concatenate([h2, ones], axis=1)
    contrib = lax.dot_general(bh_ref[...], aug, (((0,), (0,)), ((), ())),
                              preferred_element_type=jnp.float32)

    @pl.when(i == 0)
    def _():
        acc_ref[...] = contrib

    @pl.when(i > 0)
    def _():
        acc_ref[...] = acc_ref[...] + contrib

    @pl.when(i == 19)
    def _():
        a = acc_ref[...]
        pooled = a[:, 0:128] / jnp.maximum(a[:, 128:144][:, 0:1], 1.0)
        z = jnp.maximum(jnp.dot(pooled, cw1_ref[...],
                                preferred_element_type=jnp.float32)
                        + cb1_ref[...], 0.0)
        out_ref[...] = (jnp.dot(z, cw2_ref[...],
                                preferred_element_type=jnp.float32)
                        + cb2_ref[...])


def _run_k5(s2, bh, b2r, cw1, cb1r, cw2, cb2r, interpret=False):
    return pl.pallas_call(
        _k5_body,
        grid=(20,),
        in_specs=[
            pl.BlockSpec((2, 512, ROWW), lambda i: (0, i, 0)),
            pl.BlockSpec((512, NG), lambda i: (i, 0)),
            pl.BlockSpec((1, 128), lambda i: (0, 0)),
            pl.BlockSpec((128, NG), lambda i: (0, 0)),
            pl.BlockSpec((1, NG), lambda i: (0, 0)),
            pl.BlockSpec((NG, 4), lambda i: (0, 0)),
            pl.BlockSpec((1, 4), lambda i: (0, 0)),
        ],
        out_specs=pl.BlockSpec((NG, 4), lambda i: (0, 0)),
        out_shape=jax.ShapeDtypeStruct((NG, 4), jnp.float32),
        scratch_shapes=[pltpu.VMEM((NG, ROWW), jnp.float32)],
        interpret=interpret,
    )(s2, bh, b2r, cw1, cb1r, cw2, cb2r)


def kernel(x, edge_index, batch, attn_W, attn_b, W1, att_src1, att_dst1, b1,
           W2, att_src2, att_dst2, b2, cW1, cb1, cW2, cb2):
    f32 = jnp.float32
    x = x.astype(f32)
    ei = edge_index.astype(jnp.int32)
    batch32 = batch.astype(jnp.int32)

    delta = (attn_W[:, 1] - attn_W[:, 0]).reshape(IN_DIM, 1)
    wcat = jnp.concatenate(
        [W1, delta, jnp.zeros((IN_DIM, 127), f32)], axis=1)
    wcov = jnp.concatenate(
        [W1[HOG:], jnp.zeros((128 - (IN_DIM - HOG), 256), f32)], axis=0)
    covp = jnp.pad(x[:, HOG:], ((0, 0), (0, 128 - (IN_DIM - HOG))))
    att = jnp.stack([
        jnp.concatenate([att_src1[0:1, :], att_dst1[0:1, :],
                         jnp.zeros((6, 128), f32)], axis=0),
        jnp.concatenate([att_src1[1:2, :], att_dst1[1:2, :],
                         jnp.zeros((6, 128), f32)], axis=0)])
    bd = (attn_b[1] - attn_b[0]).reshape(1, 1)

    loop = jnp.arange(N, dtype=jnp.int32)
    src = jnp.concatenate([ei[0], loop, jnp.zeros((EP - E - N,), jnp.int32)])
    dst = jnp.concatenate([ei[1], loop,
                           jnp.full((EP - E - N,), N, jnp.int32)])

    att2 = jnp.concatenate([att_src2[0:1, :], att_dst2[0:1, :],
                            jnp.zeros((6, 128), f32)], axis=0)
    b1r = b1.reshape(2, 128)
    b2r = b2.reshape(1, 128)
    cb1r = cb1.reshape(1, NG)
    cb2r = cb2.reshape(1, 4)
    bh = (jnp.pad(batch32, (0, NP - N), constant_values=NG)[:, None]
          == jnp.arange(NG, dtype=jnp.int32)[None, :]).astype(f32)

    xt_heads, tabs1, m1 = _run_k1(x, covp, wcat, wcov, att, bd)
    xt_cat = xt_heads.reshape(2 * NP, ROWW)

    s1 = _make_edge_kernel(True)(xt_cat, src, dst, tabs1, m1)

    xt2, tabs2, m2 = _run_k3(s1, W2, b1r, att2)

    s2 = _make_edge_kernel(False)(xt2, src, dst, tabs2, m2)

    return _run_k5(s2, bh, b2r, cW1, cb1r, cW2, cb2r)


# speedup vs baseline: 1.1091x; 1.0658x over previous
import functools

import jax
import jax.numpy as jnp
from jax import lax
from jax.experimental import pallas as pl
from jax.experimental.pallas import tpu as pltpu
from jax.experimental.pallas import tpu_sc as plsc

N = 10000
NP = 10240
E = 320000
EP = 331776
HID = 128
ROWW = 144
NG = 64
IN_DIM = 4527
HOG = 4464
CH = 64


def _k1_body(x_ref, cov_ref, wcat_ref, wcov_ref, att_ref, bd_ref,
             xt_ref, tabs_ref, m_ref, macc_ref):
    i = pl.program_id(0)
    xb = x_ref[...]
    p = jnp.dot(xb, wcat_ref[...], preferred_element_type=jnp.float32)
    vv = jnp.dot(cov_ref[...], wcov_ref[...],
                 preferred_element_type=jnp.float32)
    q = p[:, 256:384]
    lg = q[:, 0:1] + bd_ref[0, 0]
    aw0 = 1.0 / (1.0 + jnp.exp(lg))
    xtb = aw0 * p[:, 0:256] + (1.0 - 2.0 * aw0) * vv
    row = i * 256 + lax.broadcasted_iota(jnp.int32, (256, 1), 0)
    xtb = jnp.where(row < N, xtb, 0.0)
    t0 = lax.dot_general(att_ref[0], xtb[:, 0:128], (((1,), (1,)), ((), ())),
                         preferred_element_type=jnp.float32)
    t1 = lax.dot_general(att_ref[1], xtb[:, 128:256], (((1,), (1,)), ((), ())),
                         preferred_element_type=jnp.float32)
    tabs_ref[...] = jnp.concatenate(
        [t0[1:2], t1[1:2], jnp.zeros((6, 256), jnp.float32)], axis=0)
    zero15 = jnp.zeros((256, 15), jnp.float32)
    xt_ref[...] = jnp.concatenate(
        [jnp.concatenate([xtb[:, 0:128],
                          jnp.sum(xtb[:, 0:128] * att_ref[0, 0:1, :], axis=1,
                                  keepdims=True), zero15],
                         axis=1).reshape(1, 256, ROWW),
         jnp.concatenate([xtb[:, 128:256],
                          jnp.sum(xtb[:, 128:256] * att_ref[1, 0:1, :], axis=1,
                                  keepdims=True), zero15],
                         axis=1).reshape(1, 256, ROWW)], axis=0)
    tcat = jnp.concatenate([t0[0:2], t1[0:2],
                            jnp.zeros((4, 256), jnp.float32)], axis=0)

    @pl.when(i == 0)
    def _():
        macc_ref[...] = jnp.full((8, 256), -1e30, jnp.float32)
    macc_ref[...] = jnp.maximum(macc_ref[...], tcat)

    @pl.when(i == 39)
    def _():
        mm = jnp.max(macc_ref[...], axis=1, keepdims=True)
        m0 = mm[0:1] + mm[1:2]
        m1 = mm[2:3] + mm[3:4]
        m0 = jnp.where(m0 >= 0.0, m0, 0.2 * m0)
        m1 = jnp.where(m1 >= 0.0, m1, 0.2 * m1)
        m_ref[...] = jnp.concatenate(
            [jnp.broadcast_to(m0, (1, 128)), jnp.broadcast_to(m1, (1, 128)),
             jnp.zeros((6, 128), jnp.float32)], axis=0)


def _run_k1(x, covp, wcat, wcov, att, bd, interpret=False):
    return pl.pallas_call(
        _k1_body,
        grid=(40,),
        in_specs=[
            pl.BlockSpec((256, IN_DIM), lambda i: (i, 0)),
            pl.BlockSpec((256, 128), lambda i: (i, 0)),
            pl.BlockSpec((IN_DIM, 384), lambda i: (0, 0)),
            pl.BlockSpec((128, 256), lambda i: (0, 0)),
            pl.BlockSpec((2, 8, 128), lambda i: (0, 0, 0)),
            pl.BlockSpec((1, 1), lambda i: (0, 0)),
        ],
        out_specs=[
            pl.BlockSpec((2, 256, ROWW), lambda i: (0, i, 0)),
            pl.BlockSpec((8, 256), lambda i: (0, i)),
            pl.BlockSpec((8, 128), lambda i: (0, 0)),
        ],
        out_shape=[
            jax.ShapeDtypeStruct((2, NP, ROWW), jnp.float32),
            jax.ShapeDtypeStruct((8, NP), jnp.float32),
            jax.ShapeDtypeStruct((8, 128), jnp.float32),
        ],
        scratch_shapes=[pltpu.VMEM((8, 256), jnp.float32)],
        interpret=interpret,
    )(x, covp, wcat, wcov, att, bd)


@functools.lru_cache(maxsize=None)
def _make_edge_kernel(head_split):
    n_tiles = 16 if head_split else 32
    nchunks = EP // (n_tiles * CH)
    mesh = plsc.VectorSubcoreMesh(core_axis_name="c", subcore_axis_name="s",
                                  num_cores=2, num_subcores=16)

    def body(xt_hbm, sd_hbm, tabs_hbm, m_hbm, out_hbm,
             adst_t, mvb, sdbuf, idxb, exb, gbuf, obuf, acc_sh, gsem):
        cid = lax.axis_index("c")
        sid = lax.axis_index("s")
        if head_split:
            adst_row = cid
            m_row = cid
            tile_chunk0 = sid * nchunks
            goff = cid * NP
        else:
            adst_row = 0
            m_row = 0
            wid = sid * 2 + cid
            tile_chunk0 = wid * nchunks
            goff = 0

        pltpu.sync_copy(tabs_hbm.at[adst_row], adst_t)
        pltpu.sync_copy(m_hbm.at[m_row, pl.ds(0, 16)], mvb)
        mv = mvb[...]

        def zbody(r, _):
            for q in range(ROWW // 16):
                obuf[r, pl.ds(q * 16, 16)] = jnp.zeros((16,), jnp.float32)
            return 0
        lax.fori_loop(0, 32, zbody, 0)
        rows_per_tile = NP // 16
        for r in range(rows_per_tile // 32):
            pltpu.sync_copy(obuf.at[pl.ds(0, 32)],
                            acc_sh.at[pl.ds(sid * rows_per_tile + r * 32, 32)])
        plsc.subcore_barrier()

        lane = lax.iota(jnp.int32, 16)
        col128 = jnp.full((16,), 128, jnp.int32)

        def chunk_body(c, _):
            pltpu.sync_copy(sd_hbm.at[tile_chunk0 + c], sdbuf)
            if head_split:
                def oidx(j, _):
                    idxb[pl.ds(j * 16, 16)] = (sdbuf[0, pl.ds(j * 16, 16)]
                                               + goff)
                    return 0
                lax.fori_loop(0, CH // 16, oidx, 0)
                gidx = idxb
            else:
                gidx = sdbuf.at[0]
            pltpu.async_copy(xt_hbm.at[gidx], gbuf, gsem).wait()

            def exg(j, _):
                el16 = lane + j * 16
                d16 = sdbuf[1, pl.ds(j * 16, 16)]
                a = (plsc.load_gather(gbuf, [el16, col128])
                     + plsc.load_gather(adst_t, [d16]))
                a = jnp.where(a >= 0.0, a, 0.2 * a)
                exb[pl.ds(j * 16, 16)] = jnp.exp(a - mv)
                return 0
            lax.fori_loop(0, CH // 16, exg, 0)

            def erow(j, _):
                exv = exb[pl.ds(j * 16, 16)]
                for t in range(16):
                    e = j * 16 + t
                    sp = jnp.full((16,), exv[t])
                    for q in range(8):
                        obuf[e, pl.ds(q * 16, 16)] = (
                            gbuf[e, pl.ds(q * 16, 16)] * sp)
                    obuf[e, pl.ds(128, 16)] = jnp.where(lane == 0, sp, 0.0)
                return 0
            lax.fori_loop(0, CH // 16, erow, 0)
            pltpu.sync_copy(obuf, acc_sh.at[sdbuf.at[1]], add=True)
            return 0

        lax.fori_loop(0, nchunks, chunk_body, 0)
        plsc.subcore_barrier()

        def drain(r, _):
            r0 = sid * rows_per_tile + r * 64
            pltpu.sync_copy(acc_sh.at[pl.ds(r0, 64)],
                            out_hbm.at[cid, pl.ds(r0, 64)])
            return 0
        lax.fori_loop(0, rows_per_tile // 64, drain, 0)

    return pl.kernel(
        body,
        out_type=jax.ShapeDtypeStruct((2, NP, ROWW), jnp.float32),
        mesh=mesh,
        compiler_params=pltpu.CompilerParams(needs_layout_passes=False,
                                             use_tc_tiling_on_sc=False,
                                             internal_scratch_in_bytes=32768),
        scratch_types=[
            pltpu.VMEM((NP,), jnp.float32),
            pltpu.VMEM((16,), jnp.float32),
            pltpu.VMEM((2, CH), jnp.int32),
            pltpu.VMEM((CH,), jnp.int32),
            pltpu.VMEM((CH,), jnp.float32),
            pltpu.VMEM((CH, ROWW), jnp.float32),
            pltpu.VMEM((CH, ROWW), jnp.float32),
            pltpu.VMEM_SHARED((NP, ROWW), jnp.float32),
            pltpu.SemaphoreType.DMA,
        ],
    )


def _k3_body(s_ref, w2_ref, b1_ref, att2_ref, xt2_ref, tabs2_ref, m2_ref,
             macc_ref):
    i = pl.program_id(0)
    s = s_ref[...]
    row = i * 512 + lax.broadcasted_iota(jnp.int32, (512, 1), 0)
    ok = row < N

    def head(h):
        d = s[h, :, 128:144][:, 0:1] + 1e-16
        hh = jnp.maximum(s[h, :, 0:128] / d + b1_ref[h:h + 1, :], 0.0)
        return jnp.where(ok, hh, 0.0)
    h0 = head(0)
    h1 = head(1)
    xt2b = (jnp.dot(h0, w2_ref[0:128, :], preferred_element_type=jnp.float32)
            + jnp.dot(h1, w2_ref[128:256, :],
                      preferred_element_type=jnp.float32))
    t = lax.dot_general(att2_ref[...], xt2b, (((1,), (1,)), ((), ())),
                        preferred_element_type=jnp.float32)
    tabs2_ref[...] = jnp.concatenate(
        [t[1:2], jnp.zeros((7, 512), jnp.float32)], axis=0)
    asrc2 = jnp.sum(xt2b * att2_ref[0:1, :], axis=1, keepdims=True)
    xt2_ref[...] = jnp.concatenate(
        [xt2b, asrc2, jnp.zeros((512, 15), jnp.float32)], axis=1)

    @pl.when(i == 0)
    def _():
        macc_ref[...] = jnp.full((8, 512), -1e30, jnp.float32)
    macc_ref[...] = jnp.maximum(macc_ref[...], t)

    @pl.when(i == 19)
    def _():
        mm = jnp.max(macc_ref[...], axis=1, keepdims=True)
        m0 = mm[0:1] + mm[1:2]
        m0 = jnp.where(m0 >= 0.0, m0, 0.2 * m0)
        m2_ref[...] = jnp.concatenate(
            [jnp.broadcast_to(m0, (1, 128)),
             jnp.zeros((7, 128), jnp.float32)], axis=0)


def _run_k3(s1, w2, b1r, att2, interpret=False):
    return pl.pallas_call(
        _k3_body,
        grid=(20,),
        in_specs=[
            pl.BlockSpec((2, 512, ROWW), lambda i: (0, i, 0)),
            pl.BlockSpec((256, 128), lambda i: (0, 0)),
            pl.BlockSpec((2, 128), lambda i: (0, 0)),
            pl.BlockSpec((8, 128), lambda i: (0, 0)),
        ],
        out_specs=[
            pl.BlockSpec((512, ROWW), lambda i: (i, 0)),
            pl.BlockSpec((8, 512), lambda i: (0, i)),
            pl.BlockSpec((8, 128), lambda i: (0, 0)),
        ],
        out_shape=[
            jax.ShapeDtypeStruct((NP, ROWW), jnp.float32),
            jax.ShapeDtypeStruct((8, NP), jnp.float32),
            jax.ShapeDtypeStruct((8, 128), jnp.float32),
        ],
        scratch_shapes=[pltpu.VMEM((8, 512), jnp.float32)],
        interpret=interpret,
    )(s1, w2, b1r, att2)


def _k5_body(s_ref, bh_ref, b2_ref, cw1_ref, cb1_ref, cw2_ref, cb2_ref,
             out_ref, acc_ref):
    i = pl.program_id(0)
    s = s_ref[0] + s_ref[1]
    row = i * 512 + lax.broadcasted_iota(jnp.int32, (512, 1), 0)
    ok = row < N
    d = s[:, 128:144][:, 0:1] + 1e-16
    h2 = jnp.maximum(s[:, 0:128] / d + b2_ref[...], 0.0)
    h2 = jnp.where(ok, h2, 0.0)
    lane = lax.broadcasted_iota(jnp.int32, (512, 16), 1)
    ones = jnp.where(ok & (lane == 0), 1.0, 0.0)
    aug = jnp.concatenate([h2, ones], axis=1)
    contrib = lax.dot_general(bh_ref[...], aug, (((0,), (0,)), ((), ())),
                              preferred_element_type=jnp.float32)

    @pl.when(i == 0)
    def _():
        acc_ref[...] = contrib

    @pl.when(i > 0)
    def _():
        acc_ref[...] = acc_ref[...] + contrib

    @pl.when(i == 19)
    def _():
        a = acc_ref[...]
        pooled = a[:, 0:128] / jnp.maximum(a[:, 128:144][:, 0:1], 1.0)
        z = jnp.maximum(jnp.dot(pooled, cw1_ref[...],
                                preferred_element_type=jnp.float32)
                        + cb1_ref[...], 0.0)
        out_ref[...] = (jnp.dot(z, cw2_ref[...],
                                preferred_element_type=jnp.float32)
                        + cb2_ref[...])


def _run_k5(s2, bh, b2r, cw1, cb1r, cw2, cb2r, interpret=False):
    return pl.pallas_call(
        _k5_body,
        grid=(20,),
        in_specs=[
            pl.BlockSpec((2, 512, ROWW), lambda i: (0, i, 0)),
            pl.BlockSpec((512, NG), lambda i: (i, 0)),
            pl.BlockSpec((1, 128), lambda i: (0, 0)),
            pl.BlockSpec((128, NG), lambda i: (0, 0)),
            pl.BlockSpec((1, NG), lambda i: (0, 0)),
            pl.BlockSpec((NG, 4), lambda i: (0, 0)),
            pl.BlockSpec((1, 4), lambda i: (0, 0)),
        ],
        out_specs=pl.BlockSpec((NG, 4), lambda i: (0, 0)),
        out_shape=jax.ShapeDtypeStruct((NG, 4), jnp.float32),
        scratch_shapes=[pltpu.VMEM((NG, ROWW), jnp.float32)],
        interpret=interpret,
    )(s2, bh, b2r, cw1, cb1r, cw2, cb2r)


def kernel(x, edge_index, batch, attn_W, attn_b, W1, att_src1, att_dst1, b1,
           W2, att_src2, att_dst2, b2, cW1, cb1, cW2, cb2):
    f32 = jnp.float32
    x = x.astype(f32)
    ei = edge_index.astype(jnp.int32)
    batch32 = batch.astype(jnp.int32)

    delta = (attn_W[:, 1] - attn_W[:, 0]).reshape(IN_DIM, 1)
    wcat = jnp.concatenate(
        [W1, delta, jnp.zeros((IN_DIM, 127), f32)], axis=1)
    wcov = jnp.concatenate(
        [W1[HOG:], jnp.zeros((128 - (IN_DIM - HOG), 256), f32)], axis=0)
    covp = jnp.pad(x[:, HOG:], ((0, 0), (0, 128 - (IN_DIM - HOG))))
    att = jnp.stack([
        jnp.concatenate([att_src1[0:1, :], att_dst1[0:1, :],
                         jnp.zeros((6, 128), f32)], axis=0),
        jnp.concatenate([att_src1[1:2, :], att_dst1[1:2, :],
                         jnp.zeros((6, 128), f32)], axis=0)])
    bd = (attn_b[1] - attn_b[0]).reshape(1, 1)

    loop = jnp.arange(N, dtype=jnp.int32)
    src = jnp.concatenate([ei[0], loop, jnp.zeros((EP - E - N,), jnp.int32)])
    dst = jnp.concatenate([ei[1], loop,
                           jnp.full((EP - E - N,), N, jnp.int32)])
    sd = jnp.stack([src.reshape(EP // CH, CH), dst.reshape(EP // CH, CH)],
                   axis=1)

    att2 = jnp.concatenate([att_src2[0:1, :], att_dst2[0:1, :],
                            jnp.zeros((6, 128), f32)], axis=0)
    b1r = b1.reshape(2, 128)
    b2r = b2.reshape(1, 128)
    cb1r = cb1.reshape(1, NG)
    cb2r = cb2.reshape(1, 4)
    bh = (jnp.pad(batch32, (0, NP - N), constant_values=NG)[:, None]
          == jnp.arange(NG, dtype=jnp.int32)[None, :]).astype(f32)

    xt_heads, tabs1, m1 = _run_k1(x, covp, wcat, wcov, att, bd)
    xt_cat = xt_heads.reshape(2 * NP, ROWW)

    s1 = _make_edge_kernel(True)(xt_cat, sd, tabs1, m1)

    xt2, tabs2, m2 = _run_k3(s1, W2, b1r, att2)

    s2 = _make_edge_kernel(False)(xt2, sd, tabs2, m2)

    return _run_k5(s2, bh, b2r, cW1, cb1r, cW2, cb2r)


# speedup vs baseline: 1.2263x; 1.1057x over previous
import functools

import jax
import jax.numpy as jnp
from jax import lax
from jax.experimental import pallas as pl
from jax.experimental.pallas import tpu as pltpu
from jax.experimental.pallas import tpu_sc as plsc

N = 10000
NP = 10240
E = 320000
EP = 331776
HID = 128
ROWW = 144
NG = 64
IN_DIM = 4527
HOG = 4464
CH = 64


def _k1_body(x_ref, cov_ref, wcat_ref, wcov_ref, att_ref, bd_ref,
             xt_ref, tabs_ref, m_ref, macc_ref):
    i = pl.program_id(0)
    xb = x_ref[...]
    p = jnp.dot(xb, wcat_ref[...], preferred_element_type=jnp.float32)
    vv = jnp.dot(cov_ref[...], wcov_ref[...],
                 preferred_element_type=jnp.float32)
    q = p[:, 256:384]
    lg = q[:, 0:1] + bd_ref[0, 0]
    aw0 = 1.0 / (1.0 + jnp.exp(lg))
    xtb = aw0 * p[:, 0:256] + (1.0 - 2.0 * aw0) * vv
    row = i * 256 + lax.broadcasted_iota(jnp.int32, (256, 1), 0)
    xtb = jnp.where(row < N, xtb, 0.0)
    t0 = lax.dot_general(att_ref[0], xtb[:, 0:128], (((1,), (1,)), ((), ())),
                         preferred_element_type=jnp.float32)
    t1 = lax.dot_general(att_ref[1], xtb[:, 128:256], (((1,), (1,)), ((), ())),
                         preferred_element_type=jnp.float32)
    tabs_ref[...] = jnp.concatenate(
        [t0[1:2], t1[1:2], jnp.zeros((6, 256), jnp.float32)], axis=0)
    zero15 = jnp.zeros((256, 15), jnp.float32)
    xt_ref[...] = jnp.concatenate(
        [jnp.concatenate([xtb[:, 0:128],
                          jnp.sum(xtb[:, 0:128] * att_ref[0, 0:1, :], axis=1,
                                  keepdims=True), zero15],
                         axis=1).reshape(1, 256, ROWW),
         jnp.concatenate([xtb[:, 128:256],
                          jnp.sum(xtb[:, 128:256] * att_ref[1, 0:1, :], axis=1,
                                  keepdims=True), zero15],
                         axis=1).reshape(1, 256, ROWW)], axis=0)
    tcat = jnp.concatenate([t0[0:2], t1[0:2],
                            jnp.zeros((4, 256), jnp.float32)], axis=0)

    @pl.when(i == 0)
    def _():
        macc_ref[...] = jnp.full((8, 256), -1e30, jnp.float32)
    macc_ref[...] = jnp.maximum(macc_ref[...], tcat)

    @pl.when(i == 39)
    def _():
        mm = jnp.max(macc_ref[...], axis=1, keepdims=True)
        m0 = mm[0:1] + mm[1:2]
        m1 = mm[2:3] + mm[3:4]
        m0 = jnp.where(m0 >= 0.0, m0, 0.2 * m0)
        m1 = jnp.where(m1 >= 0.0, m1, 0.2 * m1)
        m_ref[...] = jnp.concatenate(
            [jnp.broadcast_to(m0, (1, 128)), jnp.broadcast_to(m1, (1, 128)),
             jnp.zeros((6, 128), jnp.float32)], axis=0)


def _run_k1(x, covp, wcat, wcov, att, bd, interpret=False):
    return pl.pallas_call(
        _k1_body,
        grid=(40,),
        in_specs=[
            pl.BlockSpec((256, IN_DIM), lambda i: (i, 0)),
            pl.BlockSpec((256, 128), lambda i: (i, 0)),
            pl.BlockSpec((IN_DIM, 384), lambda i: (0, 0)),
            pl.BlockSpec((128, 256), lambda i: (0, 0)),
            pl.BlockSpec((2, 8, 128), lambda i: (0, 0, 0)),
            pl.BlockSpec((1, 1), lambda i: (0, 0)),
        ],
        out_specs=[
            pl.BlockSpec((2, 256, ROWW), lambda i: (0, i, 0)),
            pl.BlockSpec((8, 256), lambda i: (0, i)),
            pl.BlockSpec((8, 128), lambda i: (0, 0)),
        ],
        out_shape=[
            jax.ShapeDtypeStruct((2, NP, ROWW), jnp.float32),
            jax.ShapeDtypeStruct((8, NP), jnp.float32),
            jax.ShapeDtypeStruct((8, 128), jnp.float32),
        ],
        scratch_shapes=[pltpu.VMEM((8, 256), jnp.float32)],
        interpret=interpret,
    )(x, covp, wcat, wcov, att, bd)


@functools.lru_cache(maxsize=None)
def _make_edge_kernel(head_split):
    n_tiles = 16 if head_split else 32
    nchunks = EP // (n_tiles * CH)
    mesh = plsc.VectorSubcoreMesh(core_axis_name="c", subcore_axis_name="s",
                                  num_cores=2, num_subcores=16)

    def body(xt_hbm, sd_hbm, tabs_hbm, m_hbm, out_hbm,
             adst_t, mvb, sdbuf, idxb, sdst, exb, gbuf, obuf, acc_sh,
             gsem, ssem):
        cid = lax.axis_index("c")
        sid = lax.axis_index("s")
        if head_split:
            adst_row = cid
            m_row = cid
            tile_chunk0 = sid * nchunks
            goff = cid * NP
        else:
            adst_row = 0
            m_row = 0
            wid = sid * 2 + cid
            tile_chunk0 = wid * nchunks
            goff = 0

        pltpu.sync_copy(tabs_hbm.at[adst_row], adst_t)
        pltpu.sync_copy(m_hbm.at[m_row, pl.ds(0, 16)], mvb)
        mv = mvb[...]

        def zbody(r, _):
            for q in range(ROWW // 16):
                obuf[r, pl.ds(q * 16, 16)] = jnp.zeros((16,), jnp.float32)
            return 0
        lax.fori_loop(0, 32, zbody, 0)
        rows_per_tile = NP // 16
        for r in range(rows_per_tile // 32):
            pltpu.sync_copy(obuf.at[pl.ds(0, 32)],
                            acc_sh.at[pl.ds(sid * rows_per_tile + r * 32, 32)])
        plsc.subcore_barrier()

        lane = lax.iota(jnp.int32, 16)
        col128 = jnp.full((16,), 128, jnp.int32)

        def chunk_body(c, _):
            pltpu.sync_copy(sd_hbm.at[tile_chunk0 + c], sdbuf)
            if head_split:
                def oidx(j, _):
                    idxb[pl.ds(j * 16, 16)] = (sdbuf[0, pl.ds(j * 16, 16)]
                                               + goff)
                    return 0
                lax.fori_loop(0, CH // 16, oidx, 0)
                gidx = idxb
            else:
                gidx = sdbuf.at[0]
            pltpu.async_copy(xt_hbm.at[gidx], gbuf, gsem).wait()

            def exg(j, _):
                el16 = lane + j * 16
                d16 = sdbuf[1, pl.ds(j * 16, 16)]
                a = (plsc.load_gather(gbuf, [el16, col128])
                     + plsc.load_gather(adst_t, [d16]))
                a = jnp.where(a >= 0.0, a, 0.2 * a)
                exb[pl.ds(j * 16, 16)] = jnp.exp(a - mv)
                return 0
            lax.fori_loop(0, CH // 16, exg, 0)

            @pl.when(c >= 1)
            def _():
                pltpu.make_async_copy(obuf, acc_sh.at[sdst], ssem).wait()

            def cpd(j, _):
                sdst[pl.ds(j * 16, 16)] = sdbuf[1, pl.ds(j * 16, 16)]
                return 0
            lax.fori_loop(0, CH // 16, cpd, 0)

            def erow(j, _):
                exv = exb[pl.ds(j * 16, 16)]
                for t in range(16):
                    e = j * 16 + t
                    sp = jnp.full((16,), exv[t])
                    for q in range(8):
                        obuf[e, pl.ds(q * 16, 16)] = (
                            gbuf[e, pl.ds(q * 16, 16)] * sp)
                    obuf[e, pl.ds(128, 16)] = jnp.where(lane == 0, sp, 0.0)
                return 0
            lax.fori_loop(0, CH // 16, erow, 0)
            pltpu.async_copy(obuf, acc_sh.at[sdst], ssem, add=True)
            return 0

        lax.fori_loop(0, nchunks, chunk_body, 0)
        pltpu.make_async_copy(obuf, acc_sh.at[sdst], ssem).wait()
        plsc.subcore_barrier()

        def drain(r, _):
            r0 = sid * rows_per_tile + r * 64
            pltpu.sync_copy(acc_sh.at[pl.ds(r0, 64)],
                            out_hbm.at[cid, pl.ds(r0, 64)])
            return 0
        lax.fori_loop(0, rows_per_tile // 64, drain, 0)

    return pl.kernel(
        body,
        out_type=jax.ShapeDtypeStruct((2, NP, ROWW), jnp.float32),
        mesh=mesh,
        compiler_params=pltpu.CompilerParams(needs_layout_passes=False,
                                             use_tc_tiling_on_sc=False,
                                             internal_scratch_in_bytes=32768),
        scratch_types=[
            pltpu.VMEM((NP,), jnp.float32),
            pltpu.VMEM((16,), jnp.float32),
            pltpu.VMEM((2, CH), jnp.int32),
            pltpu.VMEM((CH,), jnp.int32),
            pltpu.VMEM((CH,), jnp.int32),
            pltpu.VMEM((CH,), jnp.float32),
            pltpu.VMEM((CH, ROWW), jnp.float32),
            pltpu.VMEM((CH, ROWW), jnp.float32),
            pltpu.VMEM_SHARED((NP, ROWW), jnp.float32),
            pltpu.SemaphoreType.DMA,
            pltpu.SemaphoreType.DMA,
        ],
    )


def _k3_body(s_ref, w2_ref, b1_ref, att2_ref, xt2_ref, tabs2_ref, m2_ref,
             macc_ref):
    i = pl.program_id(0)
    s = s_ref[...]
    row = i * 512 + lax.broadcasted_iota(jnp.int32, (512, 1), 0)
    ok = row < N

    def head(h):
        d = s[h, :, 128:144][:, 0:1] + 1e-16
        hh = jnp.maximum(s[h, :, 0:128] / d + b1_ref[h:h + 1, :], 0.0)
        return jnp.where(ok, hh, 0.0)
    h0 = head(0)
    h1 = head(1)
    xt2b = (jnp.dot(h0, w2_ref[0:128, :], preferred_element_type=jnp.float32)
            + jnp.dot(h1, w2_ref[128:256, :],
                      preferred_element_type=jnp.float32))
    t = lax.dot_general(att2_ref[...], xt2b, (((1,), (1,)), ((), ())),
                        preferred_element_type=jnp.float32)
    tabs2_ref[...] = jnp.concatenate(
        [t[1:2], jnp.zeros((7, 512), jnp.float32)], axis=0)
    asrc2 = jnp.sum(xt2b * att2_ref[0:1, :], axis=1, keepdims=True)
    xt2_ref[...] = jnp.concatenate(
        [xt2b, asrc2, jnp.zeros((512, 15), jnp.float32)], axis=1)

    @pl.when(i == 0)
    def _():
        macc_ref[...] = jnp.full((8, 512), -1e30, jnp.float32)
    macc_ref[...] = jnp.maximum(macc_ref[...], t)

    @pl.when(i == 19)
    def _():
        mm = jnp.max(macc_ref[...], axis=1, keepdims=True)
        m0 = mm[0:1] + mm[1:2]
        m0 = jnp.where(m0 >= 0.0, m0, 0.2 * m0)
        m2_ref[...] = jnp.concatenate(
            [jnp.broadcast_to(m0, (1, 128)),
             jnp.zeros((7, 128), jnp.float32)], axis=0)


def _run_k3(s1, w2, b1r, att2, interpret=False):
    return pl.pallas_call(
        _k3_body,
        grid=(20,),
        in_specs=[
            pl.BlockSpec((2, 512, ROWW), lambda i: (0, i, 0)),
            pl.BlockSpec((256, 128), lambda i: (0, 0)),
            pl.BlockSpec((2, 128), lambda i: (0, 0)),
            pl.BlockSpec((8, 128), lambda i: (0, 0)),
        ],
        out_specs=[
            pl.BlockSpec((512, ROWW), lambda i: (i, 0)),
            pl.BlockSpec((8, 512), lambda i: (0, i)),
            pl.BlockSpec((8, 128), lambda i: (0, 0)),
        ],
        out_shape=[
            jax.ShapeDtypeStruct((NP, ROWW), jnp.float32),
            jax.ShapeDtypeStruct((8, NP), jnp.float32),
            jax.ShapeDtypeStruct((8, 128), jnp.float32),
        ],
        scratch_shapes=[pltpu.VMEM((8, 512), jnp.float32)],
        interpret=interpret,
    )(s1, w2, b1r, att2)


def _k5_body(s_ref, bh_ref, b2_ref, cw1_ref, cb1_ref, cw2_ref, cb2_ref,
             out_ref, acc_ref):
    i = pl.program_id(0)
    s = s_ref[0] + s_ref[1]
    row = i * 512 + lax.broadcasted_iota(jnp.int32, (512, 1), 0)
    ok = row < N
    d = s[:, 128:144][:, 0:1] + 1e-16
    h2 = jnp.maximum(s[:, 0:128] / d + b2_ref[...], 0.0)
    h2 = jnp.where(ok, h2, 0.0)
    lane = lax.broadcasted_iota(jnp.int32, (512, 16), 1)
    ones = jnp.where(ok & (lane == 0), 1.0, 0.0)
    aug = jnp.concatenate([h2, ones], axis=1)
    contrib = lax.dot_general(bh_ref[...], aug, (((0,), (0,)), ((), ())),
                              preferred_element_type=jnp.float32)

    @pl.when(i == 0)
    def _():
        acc_ref[...] = contrib

    @pl.when(i > 0)
    def _():
        acc_ref[...] = acc_ref[...] + contrib

    @pl.when(i == 19)
    def _():
        a = acc_ref[...]
        pooled = a[:, 0:128] / jnp.maximum(a[:, 128:144][:, 0:1], 1.0)
        z = jnp.maximum(jnp.dot(pooled, cw1_ref[...],
                                preferred_element_type=jnp.float32)
                        + cb1_ref[...], 0.0)
        out_ref[...] = (jnp.dot(z, cw2_ref[...],
                                preferred_element_type=jnp.float32)
                        + cb2_ref[...])


def _run_k5(s2, bh, b2r, cw1, cb1r, cw2, cb2r, interpret=False):
    return pl.pallas_call(
        _k5_body,
        grid=(20,),
        in_specs=[
            pl.BlockSpec((2, 512, ROWW), lambda i: (0, i, 0)),
            pl.BlockSpec((512, NG), lambda i: (i, 0)),
            pl.BlockSpec((1, 128), lambda i: (0, 0)),
            pl.BlockSpec((128, NG), lambda i: (0, 0)),
            pl.BlockSpec((1, NG), lambda i: (0, 0)),
            pl.BlockSpec((NG, 4), lambda i: (0, 0)),
            pl.BlockSpec((1, 4), lambda i: (0, 0)),
        ],
        out_specs=pl.BlockSpec((NG, 4), lambda i: (0, 0)),
        out_shape=jax.ShapeDtypeStruct((NG, 4), jnp.float32),
        scratch_shapes=[pltpu.VMEM((NG, ROWW), jnp.float32)],
        interpret=interpret,
    )(s2, bh, b2r, cw1, cb1r, cw2, cb2r)


def kernel(x, edge_index, batch, attn_W, attn_b, W1, att_src1, att_dst1, b1,
           W2, att_src2, att_dst2, b2, cW1, cb1, cW2, cb2):
    f32 = jnp.float32
    x = x.astype(f32)
    ei = edge_index.astype(jnp.int32)
    batch32 = batch.astype(jnp.int32)

    delta = (attn_W[:, 1] - attn_W[:, 0]).reshape(IN_DIM, 1)
    wcat = jnp.concatenate(
        [W1, delta, jnp.zeros((IN_DIM, 127), f32)], axis=1)
    wcov = jnp.concatenate(
        [W1[HOG:], jnp.zeros((128 - (IN_DIM - HOG), 256), f32)], axis=0)
    covp = jnp.pad(x[:, HOG:], ((0, 0), (0, 128 - (IN_DIM - HOG))))
    att = jnp.stack([
        jnp.concatenate([att_src1[0:1, :], att_dst1[0:1, :],
                         jnp.zeros((6, 128), f32)], axis=0),
        jnp.concatenate([att_src1[1:2, :], att_dst1[1:2, :],
                         jnp.zeros((6, 128), f32)], axis=0)])
    bd = (attn_b[1] - attn_b[0]).reshape(1, 1)

    loop = jnp.arange(N, dtype=jnp.int32)
    src = jnp.concatenate([ei[0], loop, jnp.zeros((EP - E - N,), jnp.int32)])
    dst = jnp.concatenate([ei[1], loop,
                           jnp.full((EP - E - N,), N, jnp.int32)])
    sd = jnp.stack([src.reshape(EP // CH, CH), dst.reshape(EP // CH, CH)],
                   axis=1)

    att2 = jnp.concatenate([att_src2[0:1, :], att_dst2[0:1, :],
                            jnp.zeros((6, 128), f32)], axis=0)
    b1r = b1.reshape(2, 128)
    b2r = b2.reshape(1, 128)
    cb1r = cb1.reshape(1, NG)
    cb2r = cb2.reshape(1, 4)
    bh = (jnp.pad(batch32, (0, NP - N), constant_values=NG)[:, None]
          == jnp.arange(NG, dtype=jnp.int32)[None, :]).astype(f32)

    xt_heads, tabs1, m1 = _run_k1(x, covp, wcat, wcov, att, bd)
    xt_cat = xt_heads.reshape(2 * NP, ROWW)

    s1 = _make_edge_kernel(True)(xt_cat, sd, tabs1, m1)

    xt2, tabs2, m2 = _run_k3(s1, W2, b1r, att2)

    s2 = _make_edge_kernel(False)(xt2, sd, tabs2, m2)

    return _run_k5(s2, bh, b2r, cW1, cb1r, cW2, cb2r)


# speedup vs baseline: 1.4499x; 1.1823x over previous
import functools

import jax
import jax.numpy as jnp
from jax import lax
from jax.experimental import pallas as pl
from jax.experimental.pallas import tpu as pltpu
from jax.experimental.pallas import tpu_sc as plsc

N = 10000
NP = 10240
E = 320000
EP = 331776
HID = 128
ROWW = 144
NG = 64
IN_DIM = 4527
HOG = 4464
CH = 64


def _k1_body(x_ref, cov_ref, wcat_ref, wcov_ref, att_ref, bd_ref,
             xt_ref, tabs_ref, m_ref, macc_ref):
    i = pl.program_id(0)
    xb = x_ref[...]
    p = jnp.dot(xb, wcat_ref[...], preferred_element_type=jnp.float32)
    vv = jnp.dot(cov_ref[...], wcov_ref[...],
                 preferred_element_type=jnp.float32)
    q = p[:, 256:384]
    lg = q[:, 0:1] + bd_ref[0, 0]
    aw0 = 1.0 / (1.0 + jnp.exp(lg))
    xtb = aw0 * p[:, 0:256] + (1.0 - 2.0 * aw0) * vv
    row = i * 256 + lax.broadcasted_iota(jnp.int32, (256, 1), 0)
    xtb = jnp.where(row < N, xtb, 0.0)
    t0 = lax.dot_general(att_ref[0], xtb[:, 0:128], (((1,), (1,)), ((), ())),
                         preferred_element_type=jnp.float32)
    t1 = lax.dot_general(att_ref[1], xtb[:, 128:256], (((1,), (1,)), ((), ())),
                         preferred_element_type=jnp.float32)
    tabs_ref[...] = jnp.concatenate(
        [t0[1:2], t1[1:2], jnp.zeros((6, 256), jnp.float32)], axis=0)
    zero15 = jnp.zeros((256, 15), jnp.float32)
    xt_ref[...] = jnp.concatenate(
        [jnp.concatenate([xtb[:, 0:128],
                          jnp.sum(xtb[:, 0:128] * att_ref[0, 0:1, :], axis=1,
                                  keepdims=True), zero15],
                         axis=1).reshape(1, 256, ROWW),
         jnp.concatenate([xtb[:, 128:256],
                          jnp.sum(xtb[:, 128:256] * att_ref[1, 0:1, :], axis=1,
                                  keepdims=True), zero15],
                         axis=1).reshape(1, 256, ROWW)], axis=0)
    tcat = jnp.concatenate([t0[0:2], t1[0:2],
                            jnp.zeros((4, 256), jnp.float32)], axis=0)

    @pl.when(i == 0)
    def _():
        macc_ref[...] = jnp.full((8, 256), -1e30, jnp.float32)
    macc_ref[...] = jnp.maximum(macc_ref[...], tcat)

    @pl.when(i == 39)
    def _():
        mm = jnp.max(macc_ref[...], axis=1, keepdims=True)
        m0 = mm[0:1] + mm[1:2]
        m1 = mm[2:3] + mm[3:4]
        m0 = jnp.where(m0 >= 0.0, m0, 0.2 * m0)
        m1 = jnp.where(m1 >= 0.0, m1, 0.2 * m1)
        m_ref[...] = jnp.concatenate(
            [jnp.broadcast_to(m0, (1, 128)), jnp.broadcast_to(m1, (1, 128)),
             jnp.zeros((6, 128), jnp.float32)], axis=0)


def _run_k1(x, covp, wcat, wcov, att, bd, interpret=False):
    return pl.pallas_call(
        _k1_body,
        grid=(40,),
        in_specs=[
            pl.BlockSpec((256, IN_DIM), lambda i: (i, 0)),
            pl.BlockSpec((256, 128), lambda i: (i, 0)),
            pl.BlockSpec((IN_DIM, 384), lambda i: (0, 0)),
            pl.BlockSpec((128, 256), lambda i: (0, 0)),
            pl.BlockSpec((2, 8, 128), lambda i: (0, 0, 0)),
            pl.BlockSpec((1, 1), lambda i: (0, 0)),
        ],
        out_specs=[
            pl.BlockSpec((2, 256, ROWW), lambda i: (0, i, 0)),
            pl.BlockSpec((8, 256), lambda i: (0, i)),
            pl.BlockSpec((8, 128), lambda i: (0, 0)),
        ],
        out_shape=[
            jax.ShapeDtypeStruct((2, NP, ROWW), jnp.float32),
            jax.ShapeDtypeStruct((8, NP), jnp.float32),
            jax.ShapeDtypeStruct((8, 128), jnp.float32),
        ],
        scratch_shapes=[pltpu.VMEM((8, 256), jnp.float32)],
        interpret=interpret,
    )(x, covp, wcat, wcov, att, bd)


@functools.lru_cache(maxsize=None)
def _make_edge_kernel(head_split):
    n_tiles = 16 if head_split else 32
    nchunks = EP // (n_tiles * CH)
    mesh = plsc.VectorSubcoreMesh(core_axis_name="c", subcore_axis_name="s",
                                  num_cores=2, num_subcores=16)

    def body(xt_hbm, sd_hbm, tabs_hbm, m_hbm, out_hbm,
             adst_t, mvb, sdbuf, idxb, sdst, exb, gbuf, obuf, acc_sh,
             gsem, ssem, stsem):
        cid = lax.axis_index("c")
        sid = lax.axis_index("s")
        if head_split:
            adst_row = cid
            m_row = cid
            tile_chunk0 = sid * nchunks
            goff = cid * NP
        else:
            adst_row = 0
            m_row = 0
            wid = sid * 2 + cid
            tile_chunk0 = wid * nchunks
            goff = 0

        pltpu.sync_copy(tabs_hbm.at[adst_row], adst_t)
        pltpu.sync_copy(m_hbm.at[m_row, pl.ds(0, 16)], mvb)
        mv = mvb[...]

        def zbody(r, _):
            for q in range(ROWW // 16):
                obuf[r, pl.ds(q * 16, 16)] = jnp.zeros((16,), jnp.float32)
            return 0
        lax.fori_loop(0, 32, zbody, 0)
        rows_per_tile = NP // 16
        for r in range(rows_per_tile // 32):
            pltpu.sync_copy(obuf.at[pl.ds(0, 32)],
                            acc_sh.at[pl.ds(sid * rows_per_tile + r * 32, 32)])
        plsc.subcore_barrier()

        lane = lax.iota(jnp.int32, 16)
        col128 = jnp.full((16,), 128, jnp.int32)

        pltpu.sync_copy(sd_hbm.at[tile_chunk0], sdbuf)

        def chunk_body(c, _):
            @pl.when(c >= 1)
            def _():
                pltpu.make_async_copy(sd_hbm.at[tile_chunk0], sdbuf,
                                      stsem).wait()
            if head_split:
                def oidx(j, _):
                    idxb[pl.ds(j * 16, 16)] = (sdbuf[0, pl.ds(j * 16, 16)]
                                               + goff)
                    return 0
                lax.fori_loop(0, CH // 16, oidx, 0)
                gidx = idxb
            else:
                gidx = sdbuf.at[0]
            pltpu.async_copy(xt_hbm.at[gidx], gbuf, gsem).wait()

            def exg(j, _):
                el16 = lane + j * 16
                d16 = sdbuf[1, pl.ds(j * 16, 16)]
                a = (plsc.load_gather(gbuf, [el16, col128])
                     + plsc.load_gather(adst_t, [d16]))
                a = jnp.where(a >= 0.0, a, 0.2 * a)
                exb[pl.ds(j * 16, 16)] = jnp.exp(a - mv)
                return 0
            lax.fori_loop(0, CH // 16, exg, 0)

            @pl.when(c >= 1)
            def _():
                pltpu.make_async_copy(obuf, acc_sh.at[sdst], ssem).wait()

            def cpd(j, _):
                sdst[pl.ds(j * 16, 16)] = sdbuf[1, pl.ds(j * 16, 16)]
                return 0
            lax.fori_loop(0, CH // 16, cpd, 0)

            @pl.when(c + 1 < nchunks)
            def _():
                pltpu.async_copy(sd_hbm.at[tile_chunk0 + c + 1], sdbuf, stsem)

            def erow(j, _):
                exv = exb[pl.ds(j * 16, 16)]
                for t in range(16):
                    e = j * 16 + t
                    sp = jnp.full((16,), exv[t])
                    for q in range(8):
                        obuf[e, pl.ds(q * 16, 16)] = (
                            gbuf[e, pl.ds(q * 16, 16)] * sp)
                    obuf[e, pl.ds(128, 16)] = jnp.where(lane == 0, sp, 0.0)
                return 0
            lax.fori_loop(0, CH // 16, erow, 0)
            pltpu.async_copy(obuf, acc_sh.at[sdst], ssem, add=True)
            return 0

        lax.fori_loop(0, nchunks, chunk_body, 0)
        pltpu.make_async_copy(obuf, acc_sh.at[sdst], ssem).wait()
        plsc.subcore_barrier()

        def drain(r, _):
            r0 = sid * rows_per_tile + r * 64
            pltpu.sync_copy(acc_sh.at[pl.ds(r0, 64)],
                            out_hbm.at[cid, pl.ds(r0, 64)])
            return 0
        lax.fori_loop(0, rows_per_tile // 64, drain, 0)

    return pl.kernel(
        body,
        out_type=jax.ShapeDtypeStruct((2, NP, ROWW), jnp.float32),
        mesh=mesh,
        compiler_params=pltpu.CompilerParams(needs_layout_passes=False,
                                             use_tc_tiling_on_sc=False,
                                             internal_scratch_in_bytes=32768),
        scratch_types=[
            pltpu.VMEM((NP,), jnp.float32),
            pltpu.VMEM((16,), jnp.float32),
            pltpu.VMEM((2, CH), jnp.int32),
            pltpu.VMEM((CH,), jnp.int32),
            pltpu.VMEM((CH,), jnp.int32),
            pltpu.VMEM((CH,), jnp.float32),
            pltpu.VMEM((CH, ROWW), jnp.float32),
            pltpu.VMEM((CH, ROWW), jnp.float32),
            pltpu.VMEM_SHARED((NP, ROWW), jnp.float32),
            pltpu.SemaphoreType.DMA,
            pltpu.SemaphoreType.DMA,
            pltpu.SemaphoreType.DMA,
        ],
    )


def _k3_body(s_ref, w2_ref, b1_ref, att2_ref, xt2_ref, tabs2_ref, m2_ref,
             macc_ref):
    i = pl.program_id(0)
    s = s_ref[...]
    row = i * 512 + lax.broadcasted_iota(jnp.int32, (512, 1), 0)
    ok = row < N

    def head(h):
        d = s[h, :, 128:144][:, 0:1] + 1e-16
        hh = jnp.maximum(s[h, :, 0:128] / d + b1_ref[h:h + 1, :], 0.0)
        return jnp.where(ok, hh, 0.0)
    h0 = head(0)
    h1 = head(1)
    xt2b = (jnp.dot(h0, w2_ref[0:128, :], preferred_element_type=jnp.float32)
            + jnp.dot(h1, w2_ref[128:256, :],
                      preferred_element_type=jnp.float32))
    t = lax.dot_general(att2_ref[...], xt2b, (((1,), (1,)), ((), ())),
                        preferred_element_type=jnp.float32)
    tabs2_ref[...] = jnp.concatenate(
        [t[1:2], jnp.zeros((7, 512), jnp.float32)], axis=0)
    asrc2 = jnp.sum(xt2b * att2_ref[0:1, :], axis=1, keepdims=True)
    xt2_ref[...] = jnp.concatenate(
        [xt2b, asrc2, jnp.zeros((512, 15), jnp.float32)], axis=1)

    @pl.when(i == 0)
    def _():
        macc_ref[...] = jnp.full((8, 512), -1e30, jnp.float32)
    macc_ref[...] = jnp.maximum(macc_ref[...], t)

    @pl.when(i == 19)
    def _():
        mm = jnp.max(macc_ref[...], axis=1, keepdims=True)
        m0 = mm[0:1] + mm[1:2]
        m0 = jnp.where(m0 >= 0.0, m0, 0.2 * m0)
        m2_ref[...] = jnp.concatenate(
            [jnp.broadcast_to(m0, (1, 128)),
             jnp.zeros((7, 128), jnp.float32)], axis=0)


def _run_k3(s1, w2, b1r, att2, interpret=False):
    return pl.pallas_call(
        _k3_body,
        grid=(20,),
        in_specs=[
            pl.BlockSpec((2, 512, ROWW), lambda i: (0, i, 0)),
            pl.BlockSpec((256, 128), lambda i: (0, 0)),
            pl.BlockSpec((2, 128), lambda i: (0, 0)),
            pl.BlockSpec((8, 128), lambda i: (0, 0)),
        ],
        out_specs=[
            pl.BlockSpec((512, ROWW), lambda i: (i, 0)),
            pl.BlockSpec((8, 512), lambda i: (0, i)),
            pl.BlockSpec((8, 128), lambda i: (0, 0)),
        ],
        out_shape=[
            jax.ShapeDtypeStruct((NP, ROWW), jnp.float32),
            jax.ShapeDtypeStruct((8, NP), jnp.float32),
            jax.ShapeDtypeStruct((8, 128), jnp.float32),
        ],
        scratch_shapes=[pltpu.VMEM((8, 512), jnp.float32)],
        interpret=interpret,
    )(s1, w2, b1r, att2)


def _k5_body(s_ref, bh_ref, b2_ref, cw1_ref, cb1_ref, cw2_ref, cb2_ref,
             out_ref, acc_ref):
    i = pl.program_id(0)
    s = s_ref[0] + s_ref[1]
    row = i * 512 + lax.broadcasted_iota(jnp.int32, (512, 1), 0)
    ok = row < N
    d = s[:, 128:144][:, 0:1] + 1e-16
    h2 = jnp.maximum(s[:, 0:128] / d + b2_ref[...], 0.0)
    h2 = jnp.where(ok, h2, 0.0)
    lane = lax.broadcasted_iota(jnp.int32, (512, 16), 1)
    ones = jnp.where(ok & (lane == 0), 1.0, 0.0)
    aug = jnp.concatenate([h2, ones], axis=1)
    contrib = lax.dot_general(bh_ref[...], aug, (((0,), (0,)), ((), ())),
                              preferred_element_type=jnp.float32)

    @pl.when(i == 0)
    def _():
        acc_ref[...] = contrib

    @pl.when(i > 0)
    def _():
        acc_ref[...] = acc_ref[...] + contrib

    @pl.when(i == 19)
    def _():
        a = acc_ref[...]
        pooled = a[:, 0:128] / jnp.maximum(a[:, 128:144][:, 0:1], 1.0)
        z = jnp.maximum(jnp.dot(pooled, cw1_ref[...],
                                preferred_element_type=jnp.float32)
                        + cb1_ref[...], 0.0)
        out_ref[...] = (jnp.dot(z, cw2_ref[...],
                                preferred_element_type=jnp.float32)
                        + cb2_ref[...])


def _run_k5(s2, bh, b2r, cw1, cb1r, cw2, cb2r, interpret=False):
    return pl.pallas_call(
        _k5_body,
        grid=(20,),
        in_specs=[
            pl.BlockSpec((2, 512, ROWW), lambda i: (0, i, 0)),
            pl.BlockSpec((512, NG), lambda i: (i, 0)),
            pl.BlockSpec((1, 128), lambda i: (0, 0)),
            pl.BlockSpec((128, NG), lambda i: (0, 0)),
            pl.BlockSpec((1, NG), lambda i: (0, 0)),
            pl.BlockSpec((NG, 4), lambda i: (0, 0)),
            pl.BlockSpec((1, 4), lambda i: (0, 0)),
        ],
        out_specs=pl.BlockSpec((NG, 4), lambda i: (0, 0)),
        out_shape=jax.ShapeDtypeStruct((NG, 4), jnp.float32),
        scratch_shapes=[pltpu.VMEM((NG, ROWW), jnp.float32)],
        interpret=interpret,
    )(s2, bh, b2r, cw1, cb1r, cw2, cb2r)


def kernel(x, edge_index, batch, attn_W, attn_b, W1, att_src1, att_dst1, b1,
           W2, att_src2, att_dst2, b2, cW1, cb1, cW2, cb2):
    f32 = jnp.float32
    x = x.astype(f32)
    ei = edge_index.astype(jnp.int32)
    batch32 = batch.astype(jnp.int32)

    delta = (attn_W[:, 1] - attn_W[:, 0]).reshape(IN_DIM, 1)
    wcat = jnp.concatenate(
        [W1, delta, jnp.zeros((IN_DIM, 127), f32)], axis=1)
    wcov = jnp.concatenate(
        [W1[HOG:], jnp.zeros((128 - (IN_DIM - HOG), 256), f32)], axis=0)
    covp = jnp.pad(x[:, HOG:], ((0, 0), (0, 128 - (IN_DIM - HOG))))
    att = jnp.stack([
        jnp.concatenate([att_src1[0:1, :], att_dst1[0:1, :],
                         jnp.zeros((6, 128), f32)], axis=0),
        jnp.concatenate([att_src1[1:2, :], att_dst1[1:2, :],
                         jnp.zeros((6, 128), f32)], axis=0)])
    bd = (attn_b[1] - attn_b[0]).reshape(1, 1)

    loop = jnp.arange(N, dtype=jnp.int32)
    src = jnp.concatenate([ei[0], loop, jnp.zeros((EP - E - N,), jnp.int32)])
    dst = jnp.concatenate([ei[1], loop,
                           jnp.full((EP - E - N,), N, jnp.int32)])
    sd = jnp.stack([src.reshape(EP // CH, CH), dst.reshape(EP // CH, CH)],
                   axis=1)

    att2 = jnp.concatenate([att_src2[0:1, :], att_dst2[0:1, :],
                            jnp.zeros((6, 128), f32)], axis=0)
    b1r = b1.reshape(2, 128)
    b2r = b2.reshape(1, 128)
    cb1r = cb1.reshape(1, NG)
    cb2r = cb2.reshape(1, 4)
    bh = (jnp.pad(batch32, (0, NP - N), constant_values=NG)[:, None]
          == jnp.arange(NG, dtype=jnp.int32)[None, :]).astype(f32)

    xt_heads, tabs1, m1 = _run_k1(x, covp, wcat, wcov, att, bd)
    xt_cat = xt_heads.reshape(2 * NP, ROWW)

    s1 = _make_edge_kernel(True)(xt_cat, sd, tabs1, m1)

    xt2, tabs2, m2 = _run_k3(s1, W2, b1r, att2)

    s2 = _make_edge_kernel(False)(xt2, sd, tabs2, m2)

    return _run_k5(s2, bh, b2r, cW1, cb1r, cW2, cb2r)
